# baseline (speedup 1.0000x reference)
import numpy as np
import jax
import jax.numpy as jnp
from jax import lax
from jax.experimental import pallas as pl
from jax.experimental.pallas import tpu as pltpu
from jax.experimental.pallas import tpu_sc as plsc

_N = 10000
_E = 320000
_D = 128
_H = 128
_OUT = 10
_G = 8
_NC = 2
_NS = 16
_NT = _NC * _NS
_EPT = _E // _NT
_K = 128
_EPTP = 10240
_NCHUNK = _EPTP // _K
_WR = _K // 8
_RPT = _N // _NS
_RB = 1000
_NRB = _N // _RB
_GW = 48
_NG = 3
_KBN = float(1.0 / np.sqrt(1.0 + 1e-5))


def _sc_deg_body(c_hbm, wn_hbm, out_hbm, c_all, wexp_a, wexp_b, wdeg_v, zbuf,
                 dacc, wsem_a, wsem_b):
    cid = lax.axis_index("c")
    sid = lax.axis_index("s")
    wid = cid * _NS + sid

    def zrow(i, carry):
        zbuf[i, :] = jnp.zeros((16,), jnp.float32)
        return carry

    lax.fori_loop(0, _RPT, zrow, 0)
    pltpu.sync_copy(zbuf, dacc.at[pl.ds(sid * _RPT, _RPT)])
    pltpu.sync_copy(c_hbm.at[wid], c_all)
    plsc.subcore_barrier()

    def _wsrc(i):
        return wn_hbm.at[wid, pl.ds(i * _WR, _WR)]

    def _expand(wexp_v):
        for k in range(_K):
            wdeg_v[k, :] = wexp_v[k // 8, pl.ds((k % 8) * 16, 16)]

    pltpu.async_copy(_wsrc(0), wexp_a, wsem_a)
    pltpu.async_copy(_wsrc(1), wexp_b, wsem_b)

    def chunk2(i2, carry):
        c0 = 2 * i2
        c1 = c0 + 1
        pltpu.make_async_copy(_wsrc(c0), wexp_a, wsem_a).wait()
        _expand(wexp_a)
        pltpu.sync_copy(wdeg_v, dacc.at[c_all.at[c0]], add=True)

        @pl.when(c0 + 2 < _NCHUNK)
        def _():
            pltpu.async_copy(_wsrc(c0 + 2), wexp_a, wsem_a)

        pltpu.make_async_copy(_wsrc(c1), wexp_b, wsem_b).wait()
        _expand(wexp_b)
        pltpu.sync_copy(wdeg_v, dacc.at[c_all.at[c1]], add=True)

        @pl.when(c1 + 2 < _NCHUNK)
        def _():
            pltpu.async_copy(_wsrc(c1 + 2), wexp_b, wsem_b)

        return carry

    lax.fori_loop(0, _NCHUNK // 2, chunk2, 0)
    plsc.subcore_barrier()
    pltpu.sync_copy(dacc.at[pl.ds(sid * _RPT, _RPT)], out_hbm.at[cid, sid])


_sc_deg = pl.kernel(
    _sc_deg_body,
    out_type=jax.ShapeDtypeStruct((_NC, _NS, _RPT, 16), jnp.float32),
    mesh=plsc.VectorSubcoreMesh(core_axis_name="c", subcore_axis_name="s"),
    scratch_types=[
        pltpu.VMEM((_NCHUNK, _K), jnp.int32),
        pltpu.VMEM((_WR, 128), jnp.float32),
        pltpu.VMEM((_WR, 128), jnp.float32),
        pltpu.VMEM((_K, 16), jnp.float32),
        pltpu.VMEM((_RPT, 16), jnp.float32),
        pltpu.VMEM_SHARED((_N, 16), jnp.float32),
        pltpu.SemaphoreType.DMA,
        pltpu.SemaphoreType.DMA,
    ],
    compiler_params=pltpu.CompilerParams(use_tc_tiling_on_sc=False),
)


def _scale_rows(rows_v, wexp_v):
    for k in range(_K):
        wb = wexp_v[k // 8, pl.ds((k % 8) * 16, 16)]
        for j in range(_GW // 16):
            sl = pl.ds(j * 16, 16)
            rows_v[k, sl] = rows_v[k, sl] * wb


def _sc_agg_body(sa_hbm, sb_hbm, sc_hbm, r_hbm, c_hbm, wn_hbm,
                 oa_hbm, ob_hbm, oc_hbm,
                 r_all, c_all, wexp_a, wexp_b, rows_a, rows_b, zbuf, acc,
                 gsem_a, gsem_b, wsem_a, wsem_b):
    cid = lax.axis_index("c")
    sid = lax.axis_index("s")
    wid = cid * _NS + sid

    pltpu.sync_copy(r_hbm.at[wid], r_all)
    pltpu.sync_copy(c_hbm.at[wid], c_all)

    def _wsrc(i):
        return wn_hbm.at[wid, pl.ds(i * _WR, _WR)]

    for src_hbm, out_hbm in ((sa_hbm, oa_hbm), (sb_hbm, ob_hbm),
                             (sc_hbm, oc_hbm)):
        def zrow(i, carry):
            for j in range(_GW // 16):
                zbuf[i, pl.ds(j * 16, 16)] = jnp.zeros((16,), jnp.float32)
            return carry

        lax.fori_loop(0, 125, zrow, 0)
        for q in range(_RPT // 125):
            pltpu.sync_copy(zbuf, acc.at[pl.ds(sid * _RPT + q * 125, 125)])
        plsc.subcore_barrier()

        pltpu.async_copy(src_hbm.at[r_all.at[0]], rows_a, gsem_a)
        pltpu.async_copy(_wsrc(0), wexp_a, wsem_a)
        pltpu.async_copy(src_hbm.at[r_all.at[1]], rows_b, gsem_b)
        pltpu.async_copy(_wsrc(1), wexp_b, wsem_b)

        def chunk2(i2, carry):
            c0 = 2 * i2
            c1 = c0 + 1
            pltpu.make_async_copy(src_hbm.at[r_all.at[c0]], rows_a,
                                  gsem_a).wait()
            pltpu.make_async_copy(_wsrc(c0), wexp_a, wsem_a).wait()
            _scale_rows(rows_a, wexp_a)
            pltpu.sync_copy(rows_a, acc.at[c_all.at[c0]], add=True)

            @pl.when(c0 + 2 < _NCHUNK)
            def _():
                pltpu.async_copy(src_hbm.at[r_all.at[c0 + 2]], rows_a, gsem_a)
                pltpu.async_copy(_wsrc(c0 + 2), wexp_a, wsem_a)

            pltpu.make_async_copy(src_hbm.at[r_all.at[c1]], rows_b,
                                  gsem_b).wait()
            pltpu.make_async_copy(_wsrc(c1), wexp_b, wsem_b).wait()
            _scale_rows(rows_b, wexp_b)
            pltpu.sync_copy(rows_b, acc.at[c_all.at[c1]], add=True)

            @pl.when(c1 + 2 < _NCHUNK)
            def _():
                pltpu.async_copy(src_hbm.at[r_all.at[c1 + 2]], rows_b, gsem_b)
                pltpu.async_copy(_wsrc(c1 + 2), wexp_b, wsem_b)

            return carry

        lax.fori_loop(0, _NCHUNK // 2, chunk2, 0)
        plsc.subcore_barrier()
        pltpu.sync_copy(acc.at[pl.ds(sid * _RPT, _RPT)], out_hbm.at[cid, sid])
        plsc.subcore_barrier()


_agg_out = jax.ShapeDtypeStruct((_NC, _NS, _RPT, _GW), jnp.float32)
_sc_agg = pl.kernel(
    _sc_agg_body,
    out_type=[_agg_out, _agg_out, _agg_out],
    mesh=plsc.VectorSubcoreMesh(core_axis_name="c", subcore_axis_name="s"),
    scratch_types=[
        pltpu.VMEM((_NCHUNK, _K), jnp.int32),
        pltpu.VMEM((_NCHUNK, _K), jnp.int32),
        pltpu.VMEM((_WR, 128), jnp.float32),
        pltpu.VMEM((_WR, 128), jnp.float32),
        pltpu.VMEM((_K, _GW), jnp.float32),
        pltpu.VMEM((_K, _GW), jnp.float32),
        pltpu.VMEM((125, _GW), jnp.float32),
        pltpu.VMEM_SHARED((_N, _GW), jnp.float32),
        pltpu.SemaphoreType.DMA,
        pltpu.SemaphoreType.DMA,
        pltpu.SemaphoreType.DMA,
        pltpu.SemaphoreType.DMA,
    ],
    compiler_params=pltpu.CompilerParams(use_tc_tiling_on_sc=False),
)


_EPALL = _NT * _EPTP
_EB = _EPALL // 128 // 10


def _tc_wexp_body(ea_ref, out_ref):
    w = ea_ref[...]
    w = jnp.abs(jnp.where(w == w, w, 0.0))
    for t in range(16):
        out_ref[:, t, :] = jnp.repeat(w[:, 8 * t:8 * t + 8], 16, axis=1)


def _tc_wexp(eap):
    return pl.pallas_call(
        _tc_wexp_body,
        grid=(10,),
        in_specs=[pl.BlockSpec((_EB, 128), lambda i: (i, 0))],
        out_specs=pl.BlockSpec((_EB, 16, 128), lambda i: (i, 0, 0)),
        out_shape=jax.ShapeDtypeStruct((_EPALL // 128, 16, 128), jnp.float32),
    )(eap)


def _split_groups(ts):
    rows = ts.shape[0]
    return (ts[:, :_GW], ts[:, _GW:2 * _GW],
            jnp.concatenate(
                [ts[:, 2 * _GW:], jnp.zeros((rows, 3 * _GW - _H), jnp.float32)],
                axis=1))


def _merge_groups(ga, gb, gc):
    return jnp.concatenate([ga, gb, gc[:, :_H - 2 * _GW]], axis=1)


_SPEC_RBH = pl.BlockSpec((_RB, _H), lambda i: (i, 0))
_SPEC_RBG = pl.BlockSpec((_RB, _GW), lambda i: (i, 0))
_SPEC_RB1 = pl.BlockSpec((_RB, 1), lambda i: (i, 0))
_SPEC_1H = pl.BlockSpec((1, _H), lambda i: (0, 0))
_SPEC_HH = pl.BlockSpec((_H, _H), lambda i: (0, 0))

_SDS_G = jax.ShapeDtypeStruct((_N, _GW), jnp.float32)


def _tc_first_body(x_ref, w1a_ref, w1b_ref, d0_ref, d1_ref,
                   sa_ref, sb_ref, sc_ref, dis_ref):
    xb = x_ref[...]
    m = jnp.isnan(xb)
    xc = jnp.where(m, 0.0, xb)
    t = jnp.dot(xc, w1a_ref[...], preferred_element_type=jnp.float32)
    t = t + jnp.dot(m.astype(jnp.float32), w1b_ref[...],
                    preferred_element_type=jnp.float32)
    deg = d0_ref[...] + d1_ref[...] + 1.0
    dis = lax.rsqrt(deg)
    dis_ref[...] = dis
    ga, gb, gc = _split_groups(t * dis)
    sa_ref[...] = ga
    sb_ref[...] = gb
    sc_ref[...] = gc


def _tc_first(x, w1a, w1b, d0, d1):
    return pl.pallas_call(
        _tc_first_body,
        grid=(_NRB,),
        in_specs=[
            pl.BlockSpec((_RB, _D), lambda i: (i, 0)),
            pl.BlockSpec((_D, _H), lambda i: (0, 0)),
            pl.BlockSpec((_D, _H), lambda i: (0, 0)),
            _SPEC_RB1,
            _SPEC_RB1,
        ],
        out_specs=[_SPEC_RBG, _SPEC_RBG, _SPEC_RBG, _SPEC_RB1],
        out_shape=[_SDS_G, _SDS_G, _SDS_G,
                   jax.ShapeDtypeStruct((_N, 1), jnp.float32)],
    )(x, w1a, w1b, d0, d1)


def _tc_mid_body(a0a_ref, a1a_ref, a0b_ref, a1b_ref, a0c_ref, a1c_ref,
                 spa_ref, spb_ref, spc_ref, dis_ref, b_ref, g_ref, bb_ref,
                 w_ref, sa_ref, sb_ref, sc_ref):
    dis = dis_ref[...]
    agg = _merge_groups(a0a_ref[...] + a1a_ref[...] + spa_ref[...],
                        a0b_ref[...] + a1b_ref[...] + spb_ref[...],
                        a0c_ref[...] + a1c_ref[...] + spc_ref[...])
    u = agg * dis + b_ref[...]
    h = jnp.maximum(u * _KBN * g_ref[...] + bb_ref[...], 0.0)
    t = jnp.dot(h, w_ref[...], preferred_element_type=jnp.float32)
    ga, gb, gc = _split_groups(t * dis)
    sa_ref[...] = ga
    sb_ref[...] = gb
    sc_ref[...] = gc


def _tc_mid(agg, sp, dis, b, g, bb, w):
    return pl.pallas_call(
        _tc_mid_body,
        grid=(_NRB,),
        in_specs=[_SPEC_RBG] * 6 + [_SPEC_RBG] * 3 + [
            _SPEC_RB1, _SPEC_1H, _SPEC_1H, _SPEC_1H, _SPEC_HH,
        ],
        out_specs=[_SPEC_RBG, _SPEC_RBG, _SPEC_RBG],
        out_shape=[_SDS_G, _SDS_G, _SDS_G],
    )(agg[0][0], agg[0][1], agg[1][0], agg[1][1], agg[2][0], agg[2][1],
      sp[0], sp[1], sp[2], dis, b, g, bb, w)


def _tc_final_body(a0a_ref, a1a_ref, a0b_ref, a1b_ref, a0c_ref, a1c_ref,
                   spa_ref, spb_ref, spc_ref, dis_ref, b3_ref, batch_ref,
                   mw1_ref, mb1_ref, mw2_ref, mb2_ref, out_ref, sums, cnts):
    i = pl.program_id(0)

    @pl.when(i == 0)
    def _init():
        sums[...] = jnp.zeros_like(sums)
        cnts[...] = jnp.zeros_like(cnts)

    agg = _merge_groups(a0a_ref[...] + a1a_ref[...] + spa_ref[...],
                        a0b_ref[...] + a1b_ref[...] + spb_ref[...],
                        a0c_ref[...] + a1c_ref[...] + spc_ref[...])
    h3 = agg * dis_ref[...] + b3_ref[...]
    bv = batch_ref[0]
    oh = (lax.broadcasted_iota(jnp.int32, (_G, _RB), 0) == bv).astype(
        jnp.float32)
    sums[...] += jnp.dot(oh, h3, preferred_element_type=jnp.float32)
    cnts[...] = cnts[...] + jnp.sum(oh, axis=1, keepdims=True)

    @pl.when(i == pl.num_programs(0) - 1)
    def _fin():
        hg = sums[...] / jnp.maximum(cnts[...], 1.0)
        z1 = jnp.dot(hg, mw1_ref[...], preferred_element_type=jnp.float32)
        z1 = z1 + mb1_ref[...]
        z1 = 0.5 * z1 * (1.0 + lax.erf(z1 * float(1.0 / np.sqrt(2.0))))
        out_ref[...] = jnp.dot(z1, mw2_ref[...],
                               preferred_element_type=jnp.float32) + mb2_ref[...]


def _tc_final(agg, sp, dis, b3, batch3, mw1, mb1, mw2p, mb2p):
    return pl.pallas_call(
        _tc_final_body,
        grid=(_NRB,),
        in_specs=[_SPEC_RBG] * 6 + [_SPEC_RBG] * 3 + [
            _SPEC_RB1, _SPEC_1H,
            pl.BlockSpec((1, 1, _RB), lambda i: (i, 0, 0)),
            _SPEC_HH, _SPEC_1H, _SPEC_HH, _SPEC_1H,
        ],
        out_specs=pl.BlockSpec((_G, _H), lambda i: (0, 0)),
        out_shape=jax.ShapeDtypeStruct((_G, _H), jnp.float32),
        scratch_shapes=[
            pltpu.VMEM((_G, _H), jnp.float32),
            pltpu.VMEM((_G, _H), jnp.float32),
        ],
    )(agg[0][0], agg[0][1], agg[1][0], agg[1][1], agg[2][0], agg[2][1],
      sp[0], sp[1], sp[2], dis, b3, batch3, mw1, mb1, mw2p, mb2p)


def _agg_groups(sp, r3, c3, wn):
    outs = _sc_agg(sp[0], sp[1], sp[2], r3, c3, wn)
    res = []
    for o in outs:
        o = o.reshape(_NC, _N, _GW)
        res.append((o[0], o[1]))
    return res


def kernel(x, edge_index, edge_attr, batch, W1, b1, W2, b2, W3, b3,
           bn_g, bn_b, mW1, mb1, mW2, mb2):
    pad = ((0, 0), (0, _EPTP - _EPT))
    r3 = jnp.pad(edge_index[0].astype(jnp.int32).reshape(_NT, _EPT),
                 pad).reshape(_NT, _NCHUNK, _K)
    c3 = jnp.pad(edge_index[1].astype(jnp.int32).reshape(_NT, _EPT),
                 pad).reshape(_NT, _NCHUNK, _K)
    batch3 = batch.reshape(_NRB, 1, _RB).astype(jnp.int32)

    eap = jnp.pad(edge_attr.reshape(_NT, _EPT), pad).reshape(_EPALL // 128, 128)
    wn = _tc_wexp(eap).reshape(_NT, _EPTP // 8, 128)
    degp = _sc_deg(c3, wn).reshape(_NC, _N, 16)
    d0 = degp[0, :, 0:1]
    d1 = degp[1, :, 0:1]

    sa, sb, sc, dis = _tc_first(x, W1[:_D], W1[_D:], d0, d1)
    sp = (sa, sb, sc)

    b1r = b1.reshape(1, _H)
    b2r = b2.reshape(1, _H)
    b3r = b3.reshape(1, _H)
    gr = bn_g.reshape(1, _H)
    bbr = bn_b.reshape(1, _H)

    agg = _agg_groups(sp, r3, c3, wn)
    sp = _tc_mid(agg, sp, dis, b1r, gr, bbr, W2)
    agg = _agg_groups(sp, r3, c3, wn)
    sp = _tc_mid(agg, sp, dis, b2r, gr, bbr, W3)
    agg = _agg_groups(sp, r3, c3, wn)

    mw2p = jnp.pad(mW2, ((0, 0), (0, _H - _OUT)))
    mb2p = jnp.pad(mb2.reshape(1, _OUT), ((0, 0), (0, _H - _OUT)))
    zf = _tc_final(agg, sp, dis, b3r, batch3,
                   mW1, mb1.reshape(1, _H), mw2p, mb2p)
    return zf[:, :_OUT]


# speedup vs baseline: 1.0612x; 1.0612x over previous
import numpy as np
import jax
import jax.numpy as jnp
from jax import lax
from jax.experimental import pallas as pl
from jax.experimental.pallas import tpu as pltpu
from jax.experimental.pallas import tpu_sc as plsc

_N = 10000
_E = 320000
_D = 128
_H = 128
_OUT = 10
_G = 8
_NC = 2
_NS = 16
_NT = _NC * _NS
_EPT = _E // _NT
_K = 64
_EPTP = 10240
_NCHUNK = _EPTP // _K
_WR = _K // 8
_RPT = _N // _NS
_RB = 1000
_NRB = _N // _RB
_GW = 48
_NG = 3
_KBN = float(1.0 / np.sqrt(1.0 + 1e-5))


def _sc_deg_body(c_hbm, wn_hbm, out_hbm, c_all, wexp_a, wexp_b, wdeg_v, zbuf,
                 dacc, wsem_a, wsem_b):
    cid = lax.axis_index("c")
    sid = lax.axis_index("s")
    wid = cid * _NS + sid

    def zrow(i, carry):
        zbuf[i, :] = jnp.zeros((16,), jnp.float32)
        return carry

    lax.fori_loop(0, _RPT, zrow, 0)
    pltpu.sync_copy(zbuf, dacc.at[pl.ds(sid * _RPT, _RPT)])
    pltpu.sync_copy(c_hbm.at[wid], c_all)
    plsc.subcore_barrier()

    def _wsrc(i):
        return wn_hbm.at[wid, pl.ds(i * _WR, _WR)]

    def _expand(wexp_v):
        for k in range(_K):
            wdeg_v[k, :] = wexp_v[k // 8, pl.ds((k % 8) * 16, 16)]

    pltpu.async_copy(_wsrc(0), wexp_a, wsem_a)
    pltpu.async_copy(_wsrc(1), wexp_b, wsem_b)

    def chunk2(i2, carry):
        c0 = 2 * i2
        c1 = c0 + 1
        pltpu.make_async_copy(_wsrc(c0), wexp_a, wsem_a).wait()
        _expand(wexp_a)
        pltpu.sync_copy(wdeg_v, dacc.at[c_all.at[c0]], add=True)

        @pl.when(c0 + 2 < _NCHUNK)
        def _():
            pltpu.async_copy(_wsrc(c0 + 2), wexp_a, wsem_a)

        pltpu.make_async_copy(_wsrc(c1), wexp_b, wsem_b).wait()
        _expand(wexp_b)
        pltpu.sync_copy(wdeg_v, dacc.at[c_all.at[c1]], add=True)

        @pl.when(c1 + 2 < _NCHUNK)
        def _():
            pltpu.async_copy(_wsrc(c1 + 2), wexp_b, wsem_b)

        return carry

    lax.fori_loop(0, _NCHUNK // 2, chunk2, 0)
    plsc.subcore_barrier()
    pltpu.sync_copy(dacc.at[pl.ds(sid * _RPT, _RPT)], out_hbm.at[cid, sid])


_sc_deg = pl.kernel(
    _sc_deg_body,
    out_type=jax.ShapeDtypeStruct((_NC, _NS, _RPT, 16), jnp.float32),
    mesh=plsc.VectorSubcoreMesh(core_axis_name="c", subcore_axis_name="s"),
    scratch_types=[
        pltpu.VMEM((_NCHUNK, _K), jnp.int32),
        pltpu.VMEM((_WR, 128), jnp.float32),
        pltpu.VMEM((_WR, 128), jnp.float32),
        pltpu.VMEM((_K, 16), jnp.float32),
        pltpu.VMEM((_RPT, 16), jnp.float32),
        pltpu.VMEM_SHARED((_N, 16), jnp.float32),
        pltpu.SemaphoreType.DMA,
        pltpu.SemaphoreType.DMA,
    ],
    compiler_params=pltpu.CompilerParams(use_tc_tiling_on_sc=False),
)


def _scale_rows(rows_v, wexp_v):
    for k in range(_K):
        wb = wexp_v[k // 8, pl.ds((k % 8) * 16, 16)]
        for j in range(_GW // 16):
            sl = pl.ds(j * 16, 16)
            rows_v[k, sl] = rows_v[k, sl] * wb


def _sc_agg_body(sa_hbm, sb_hbm, sc_hbm, r_hbm, c_hbm, wn_hbm,
                 oa_hbm, ob_hbm, oc_hbm,
                 r_all, c_all, wexp_a, wexp_b, rows_a, rows_b, zbuf, acc,
                 gsem_a, gsem_b, wsem_a, wsem_b):
    cid = lax.axis_index("c")
    sid = lax.axis_index("s")
    wid = cid * _NS + sid

    pltpu.sync_copy(r_hbm.at[wid], r_all)
    pltpu.sync_copy(c_hbm.at[wid], c_all)

    def _wsrc(i):
        return wn_hbm.at[wid, pl.ds(i * _WR, _WR)]

    for src_hbm, out_hbm in ((sa_hbm, oa_hbm), (sb_hbm, ob_hbm),
                             (sc_hbm, oc_hbm)):
        def zrow(i, carry):
            for j in range(_GW // 16):
                zbuf[i, pl.ds(j * 16, 16)] = jnp.zeros((16,), jnp.float32)
            return carry

        lax.fori_loop(0, 125, zrow, 0)
        for q in range(_RPT // 125):
            pltpu.sync_copy(zbuf, acc.at[pl.ds(sid * _RPT + q * 125, 125)])
        plsc.subcore_barrier()

        pltpu.async_copy(src_hbm.at[r_all.at[0]], rows_a, gsem_a)
        pltpu.async_copy(_wsrc(0), wexp_a, wsem_a)
        pltpu.async_copy(src_hbm.at[r_all.at[1]], rows_b, gsem_b)
        pltpu.async_copy(_wsrc(1), wexp_b, wsem_b)

        def chunk2(i2, carry):
            c0 = 2 * i2
            c1 = c0 + 1
            pltpu.make_async_copy(src_hbm.at[r_all.at[c0]], rows_a,
                                  gsem_a).wait()
            pltpu.make_async_copy(_wsrc(c0), wexp_a, wsem_a).wait()
            _scale_rows(rows_a, wexp_a)
            pltpu.sync_copy(rows_a, acc.at[c_all.at[c0]], add=True)

            @pl.when(c0 + 2 < _NCHUNK)
            def _():
                pltpu.async_copy(src_hbm.at[r_all.at[c0 + 2]], rows_a, gsem_a)
                pltpu.async_copy(_wsrc(c0 + 2), wexp_a, wsem_a)

            pltpu.make_async_copy(src_hbm.at[r_all.at[c1]], rows_b,
                                  gsem_b).wait()
            pltpu.make_async_copy(_wsrc(c1), wexp_b, wsem_b).wait()
            _scale_rows(rows_b, wexp_b)
            pltpu.sync_copy(rows_b, acc.at[c_all.at[c1]], add=True)

            @pl.when(c1 + 2 < _NCHUNK)
            def _():
                pltpu.async_copy(src_hbm.at[r_all.at[c1 + 2]], rows_b, gsem_b)
                pltpu.async_copy(_wsrc(c1 + 2), wexp_b, wsem_b)

            return carry

        lax.fori_loop(0, _NCHUNK // 2, chunk2, 0)
        plsc.subcore_barrier()
        pltpu.sync_copy(acc.at[pl.ds(sid * _RPT, _RPT)], out_hbm.at[cid, sid])
        plsc.subcore_barrier()


_agg_out = jax.ShapeDtypeStruct((_NC, _NS, _RPT, _GW), jnp.float32)
_sc_agg = pl.kernel(
    _sc_agg_body,
    out_type=[_agg_out, _agg_out, _agg_out],
    mesh=plsc.VectorSubcoreMesh(core_axis_name="c", subcore_axis_name="s"),
    scratch_types=[
        pltpu.VMEM((_NCHUNK, _K), jnp.int32),
        pltpu.VMEM((_NCHUNK, _K), jnp.int32),
        pltpu.VMEM((_WR, 128), jnp.float32),
        pltpu.VMEM((_WR, 128), jnp.float32),
        pltpu.VMEM((_K, _GW), jnp.float32),
        pltpu.VMEM((_K, _GW), jnp.float32),
        pltpu.VMEM((125, _GW), jnp.float32),
        pltpu.VMEM_SHARED((_N, _GW), jnp.float32),
        pltpu.SemaphoreType.DMA,
        pltpu.SemaphoreType.DMA,
        pltpu.SemaphoreType.DMA,
        pltpu.SemaphoreType.DMA,
    ],
    compiler_params=pltpu.CompilerParams(use_tc_tiling_on_sc=False),
)


_EPALL = _NT * _EPTP
_EB = _EPALL // 128 // 10


def _tc_wexp_body(ea_ref, m_ref, out_ref):
    w = ea_ref[...]
    w = jnp.abs(jnp.where(w == w, w, 0.0))
    out_ref[...] = jnp.dot(w, m_ref[...], preferred_element_type=jnp.float32)


def _tc_wexp(eap, m):
    return pl.pallas_call(
        _tc_wexp_body,
        grid=(10,),
        in_specs=[
            pl.BlockSpec((_EB, 128), lambda i: (i, 0)),
            pl.BlockSpec((128, 2048), lambda i: (0, 0)),
        ],
        out_specs=pl.BlockSpec((_EB, 2048), lambda i: (i, 0)),
        out_shape=jax.ShapeDtypeStruct((_EPALL // 128, 2048), jnp.float32),
    )(eap, m)


def _split_groups(ts):
    rows = ts.shape[0]
    return (ts[:, :_GW], ts[:, _GW:2 * _GW],
            jnp.concatenate(
                [ts[:, 2 * _GW:], jnp.zeros((rows, 3 * _GW - _H), jnp.float32)],
                axis=1))


def _merge_groups(ga, gb, gc):
    return jnp.concatenate([ga, gb, gc[:, :_H - 2 * _GW]], axis=1)


_SPEC_RBH = pl.BlockSpec((_RB, _H), lambda i: (i, 0))
_SPEC_RBG = pl.BlockSpec((_RB, _GW), lambda i: (i, 0))
_SPEC_RB1 = pl.BlockSpec((_RB, 1), lambda i: (i, 0))
_SPEC_1H = pl.BlockSpec((1, _H), lambda i: (0, 0))
_SPEC_HH = pl.BlockSpec((_H, _H), lambda i: (0, 0))

_SDS_G = jax.ShapeDtypeStruct((_N, _GW), jnp.float32)


def _tc_first_body(x_ref, w1a_ref, w1b_ref, d0_ref, d1_ref,
                   sa_ref, sb_ref, sc_ref, dis_ref):
    xb = x_ref[...]
    m = jnp.isnan(xb)
    xc = jnp.where(m, 0.0, xb)
    t = jnp.dot(xc, w1a_ref[...], preferred_element_type=jnp.float32)
    t = t + jnp.dot(m.astype(jnp.float32), w1b_ref[...],
                    preferred_element_type=jnp.float32)
    deg = d0_ref[...] + d1_ref[...] + 1.0
    dis = lax.rsqrt(deg)
    dis_ref[...] = dis
    ga, gb, gc = _split_groups(t * dis)
    sa_ref[...] = ga
    sb_ref[...] = gb
    sc_ref[...] = gc


def _tc_first(x, w1a, w1b, d0, d1):
    return pl.pallas_call(
        _tc_first_body,
        grid=(_NRB,),
        in_specs=[
            pl.BlockSpec((_RB, _D), lambda i: (i, 0)),
            pl.BlockSpec((_D, _H), lambda i: (0, 0)),
            pl.BlockSpec((_D, _H), lambda i: (0, 0)),
            _SPEC_RB1,
            _SPEC_RB1,
        ],
        out_specs=[_SPEC_RBG, _SPEC_RBG, _SPEC_RBG, _SPEC_RB1],
        out_shape=[_SDS_G, _SDS_G, _SDS_G,
                   jax.ShapeDtypeStruct((_N, 1), jnp.float32)],
    )(x, w1a, w1b, d0, d1)


def _tc_mid_body(a0a_ref, a1a_ref, a0b_ref, a1b_ref, a0c_ref, a1c_ref,
                 spa_ref, spb_ref, spc_ref, dis_ref, b_ref, g_ref, bb_ref,
                 w_ref, sa_ref, sb_ref, sc_ref):
    dis = dis_ref[...]
    agg = _merge_groups(a0a_ref[...] + a1a_ref[...] + spa_ref[...],
                        a0b_ref[...] + a1b_ref[...] + spb_ref[...],
                        a0c_ref[...] + a1c_ref[...] + spc_ref[...])
    u = agg * dis + b_ref[...]
    h = jnp.maximum(u * _KBN * g_ref[...] + bb_ref[...], 0.0)
    t = jnp.dot(h, w_ref[...], preferred_element_type=jnp.float32)
    ga, gb, gc = _split_groups(t * dis)
    sa_ref[...] = ga
    sb_ref[...] = gb
    sc_ref[...] = gc


def _tc_mid(agg, sp, dis, b, g, bb, w):
    return pl.pallas_call(
        _tc_mid_body,
        grid=(_NRB,),
        in_specs=[_SPEC_RBG] * 6 + [_SPEC_RBG] * 3 + [
            _SPEC_RB1, _SPEC_1H, _SPEC_1H, _SPEC_1H, _SPEC_HH,
        ],
        out_specs=[_SPEC_RBG, _SPEC_RBG, _SPEC_RBG],
        out_shape=[_SDS_G, _SDS_G, _SDS_G],
    )(agg[0][0], agg[0][1], agg[1][0], agg[1][1], agg[2][0], agg[2][1],
      sp[0], sp[1], sp[2], dis, b, g, bb, w)


def _tc_final_body(a0a_ref, a1a_ref, a0b_ref, a1b_ref, a0c_ref, a1c_ref,
                   spa_ref, spb_ref, spc_ref, dis_ref, b3_ref, batch_ref,
                   mw1_ref, mb1_ref, mw2_ref, mb2_ref, out_ref, sums, cnts):
    i = pl.program_id(0)

    @pl.when(i == 0)
    def _init():
        sums[...] = jnp.zeros_like(sums)
        cnts[...] = jnp.zeros_like(cnts)

    agg = _merge_groups(a0a_ref[...] + a1a_ref[...] + spa_ref[...],
                        a0b_ref[...] + a1b_ref[...] + spb_ref[...],
                        a0c_ref[...] + a1c_ref[...] + spc_ref[...])
    h3 = agg * dis_ref[...] + b3_ref[...]
    bv = batch_ref[0]
    oh = (lax.broadcasted_iota(jnp.int32, (_G, _RB), 0) == bv).astype(
        jnp.float32)
    sums[...] += jnp.dot(oh, h3, preferred_element_type=jnp.float32)
    cnts[...] = cnts[...] + jnp.sum(oh, axis=1, keepdims=True)

    @pl.when(i == pl.num_programs(0) - 1)
    def _fin():
        hg = sums[...] / jnp.maximum(cnts[...], 1.0)
        z1 = jnp.dot(hg, mw1_ref[...], preferred_element_type=jnp.float32)
        z1 = z1 + mb1_ref[...]
        z1 = 0.5 * z1 * (1.0 + lax.erf(z1 * float(1.0 / np.sqrt(2.0))))
        out_ref[...] = jnp.dot(z1, mw2_ref[...],
                               preferred_element_type=jnp.float32) + mb2_ref[...]


def _tc_final(agg, sp, dis, b3, batch3, mw1, mb1, mw2p, mb2p):
    return pl.pallas_call(
        _tc_final_body,
        grid=(_NRB,),
        in_specs=[_SPEC_RBG] * 6 + [_SPEC_RBG] * 3 + [
            _SPEC_RB1, _SPEC_1H,
            pl.BlockSpec((1, 1, _RB), lambda i: (i, 0, 0)),
            _SPEC_HH, _SPEC_1H, _SPEC_HH, _SPEC_1H,
        ],
        out_specs=pl.BlockSpec((_G, _H), lambda i: (0, 0)),
        out_shape=jax.ShapeDtypeStruct((_G, _H), jnp.float32),
        scratch_shapes=[
            pltpu.VMEM((_G, _H), jnp.float32),
            pltpu.VMEM((_G, _H), jnp.float32),
        ],
    )(agg[0][0], agg[0][1], agg[1][0], agg[1][1], agg[2][0], agg[2][1],
      sp[0], sp[1], sp[2], dis, b3, batch3, mw1, mb1, mw2p, mb2p)


def _agg_groups(sp, r3, c3, wn):
    outs = _sc_agg(sp[0], sp[1], sp[2], r3, c3, wn)
    res = []
    for o in outs:
        o = o.reshape(_NC, _N, _GW)
        res.append((o[0], o[1]))
    return res


def kernel(x, edge_index, edge_attr, batch, W1, b1, W2, b2, W3, b3,
           bn_g, bn_b, mW1, mb1, mW2, mb2):
    pad = ((0, 0), (0, _EPTP - _EPT))
    r3 = jnp.pad(edge_index[0].astype(jnp.int32).reshape(_NT, _EPT),
                 pad).reshape(_NT, _NCHUNK, _K)
    c3 = jnp.pad(edge_index[1].astype(jnp.int32).reshape(_NT, _EPT),
                 pad).reshape(_NT, _NCHUNK, _K)
    batch3 = batch.reshape(_NRB, 1, _RB).astype(jnp.int32)

    eap = jnp.pad(edge_attr.reshape(_NT, _EPT), pad).reshape(_EPALL // 128, 128)
    msel = (jnp.arange(128)[:, None] == (jnp.arange(2048) // 16)[None, :]
            ).astype(jnp.float32)
    wn = _tc_wexp(eap, msel).reshape(_NT, _EPTP // 8, 128)
    degp = _sc_deg(c3, wn).reshape(_NC, _N, 16)
    d0 = degp[0, :, 0:1]
    d1 = degp[1, :, 0:1]

    sa, sb, sc, dis = _tc_first(x, W1[:_D], W1[_D:], d0, d1)
    sp = (sa, sb, sc)

    b1r = b1.reshape(1, _H)
    b2r = b2.reshape(1, _H)
    b3r = b3.reshape(1, _H)
    gr = bn_g.reshape(1, _H)
    bbr = bn_b.reshape(1, _H)

    agg = _agg_groups(sp, r3, c3, wn)
    sp = _tc_mid(agg, sp, dis, b1r, gr, bbr, W2)
    agg = _agg_groups(sp, r3, c3, wn)
    sp = _tc_mid(agg, sp, dis, b2r, gr, bbr, W3)
    agg = _agg_groups(sp, r3, c3, wn)

    mw2p = jnp.pad(mW2, ((0, 0), (0, _H - _OUT)))
    mb2p = jnp.pad(mb2.reshape(1, _OUT), ((0, 0), (0, _H - _OUT)))
    zf = _tc_final(agg, sp, dis, b3r, batch3,
                   mW1, mb1.reshape(1, _H), mw2p, mb2p)
    return zf[:, :_OUT]


# speedup vs baseline: 1.9023x; 1.7926x over previous
import numpy as np
import jax
import jax.numpy as jnp
from jax import lax
from jax.experimental import pallas as pl
from jax.experimental.pallas import tpu as pltpu
from jax.experimental.pallas import tpu_sc as plsc

_N = 10000
_E = 320000
_D = 128
_H = 128
_OUT = 10
_G = 8
_NC = 2
_NS = 16
_NT = _NC * _NS
_EPT = _E // _NT
_K = 80
_EPTP = 10240
_NCHUNK = _EPTP // _K
_WR = _K // 8
_RPT = _N // _NS
_RB = 1000
_NRB = _N // _RB
_GW = 48
_NG = 3
_KBN = float(1.0 / np.sqrt(1.0 + 1e-5))


def _sc_deg_body(c_hbm, wn_hbm, out_hbm, c_all, wexp_a, wexp_b, wdeg_v, zbuf,
                 dacc, wsem_a, wsem_b):
    cid = lax.axis_index("c")
    sid = lax.axis_index("s")
    wid = cid * _NS + sid

    def zrow(i, carry):
        zbuf[i, :] = jnp.zeros((16,), jnp.float32)
        return carry

    lax.fori_loop(0, _RPT, zrow, 0)
    pltpu.sync_copy(zbuf, dacc.at[pl.ds(sid * _RPT, _RPT)])
    pltpu.sync_copy(c_hbm.at[wid], c_all)
    plsc.subcore_barrier()

    def _wsrc(i):
        return wn_hbm.at[wid, pl.ds(i * _WR, _WR)]

    def _expand(wexp_v):
        for k in range(_K):
            wdeg_v[k, :] = wexp_v[k // 8, pl.ds((k % 8) * 16, 16)]

    pltpu.async_copy(_wsrc(0), wexp_a, wsem_a)
    pltpu.async_copy(_wsrc(1), wexp_b, wsem_b)

    def chunk2(i2, carry):
        c0 = 2 * i2
        c1 = c0 + 1
        pltpu.make_async_copy(_wsrc(c0), wexp_a, wsem_a).wait()
        _expand(wexp_a)
        pltpu.sync_copy(wdeg_v, dacc.at[c_all.at[c0]], add=True)

        @pl.when(c0 + 2 < _NCHUNK)
        def _():
            pltpu.async_copy(_wsrc(c0 + 2), wexp_a, wsem_a)

        pltpu.make_async_copy(_wsrc(c1), wexp_b, wsem_b).wait()
        _expand(wexp_b)
        pltpu.sync_copy(wdeg_v, dacc.at[c_all.at[c1]], add=True)

        @pl.when(c1 + 2 < _NCHUNK)
        def _():
            pltpu.async_copy(_wsrc(c1 + 2), wexp_b, wsem_b)

        return carry

    lax.fori_loop(0, _NCHUNK // 2, chunk2, 0)
    plsc.subcore_barrier()
    pltpu.sync_copy(dacc.at[pl.ds(sid * _RPT, _RPT)], out_hbm.at[cid, sid])


_sc_deg = pl.kernel(
    _sc_deg_body,
    out_type=jax.ShapeDtypeStruct((_NC, _NS, _RPT, 16), jnp.float32),
    mesh=plsc.VectorSubcoreMesh(core_axis_name="c", subcore_axis_name="s"),
    scratch_types=[
        pltpu.VMEM((_NCHUNK, _K), jnp.int32),
        pltpu.VMEM((_WR, 128), jnp.float32),
        pltpu.VMEM((_WR, 128), jnp.float32),
        pltpu.VMEM((_K, 16), jnp.float32),
        pltpu.VMEM((_RPT, 16), jnp.float32),
        pltpu.VMEM_SHARED((_N, 16), jnp.float32),
        pltpu.SemaphoreType.DMA,
        pltpu.SemaphoreType.DMA,
    ],
    compiler_params=pltpu.CompilerParams(use_tc_tiling_on_sc=False),
)


def _scale_rows(rows_v, wexp_v):
    for k in range(_K):
        wb = wexp_v[k // 8, pl.ds((k % 8) * 16, 16)]
        for j in range(_GW // 16):
            sl = pl.ds(j * 16, 16)
            rows_v[k, sl] = rows_v[k, sl] * wb


def _sc_agg_body(sa_hbm, sb_hbm, sc_hbm, r_hbm, c_hbm, wn_hbm,
                 oa_hbm, ob_hbm, oc_hbm,
                 r_all, c_all, wexp_a, wexp_b, rows_a, rows_b, zbuf, acc,
                 gsem_a, gsem_b, wsem_a, wsem_b):
    cid = lax.axis_index("c")
    sid = lax.axis_index("s")
    wid = cid * _NS + sid

    pltpu.sync_copy(r_hbm.at[wid], r_all)
    pltpu.sync_copy(c_hbm.at[wid], c_all)

    def _wsrc(i):
        return wn_hbm.at[wid, pl.ds(i * _WR, _WR)]

    for src_hbm, out_hbm in ((sa_hbm, oa_hbm), (sb_hbm, ob_hbm),
                             (sc_hbm, oc_hbm)):
        def zrow(i, carry):
            for j in range(_GW // 16):
                zbuf[i, pl.ds(j * 16, 16)] = jnp.zeros((16,), jnp.float32)
            return carry

        lax.fori_loop(0, 125, zrow, 0)
        for q in range(_RPT // 125):
            pltpu.sync_copy(zbuf, acc.at[pl.ds(sid * _RPT + q * 125, 125)])
        plsc.subcore_barrier()

        pltpu.async_copy(src_hbm.at[r_all.at[0]], rows_a, gsem_a)
        pltpu.async_copy(_wsrc(0), wexp_a, wsem_a)
        pltpu.async_copy(src_hbm.at[r_all.at[1]], rows_b, gsem_b)
        pltpu.async_copy(_wsrc(1), wexp_b, wsem_b)

        def chunk2(i2, carry):
            c0 = 2 * i2
            c1 = c0 + 1
            pltpu.make_async_copy(src_hbm.at[r_all.at[c0]], rows_a,
                                  gsem_a).wait()
            pltpu.make_async_copy(_wsrc(c0), wexp_a, wsem_a).wait()
            _scale_rows(rows_a, wexp_a)
            pltpu.sync_copy(rows_a, acc.at[c_all.at[c0]], add=True)

            @pl.when(c0 + 2 < _NCHUNK)
            def _():
                pltpu.async_copy(src_hbm.at[r_all.at[c0 + 2]], rows_a, gsem_a)
                pltpu.async_copy(_wsrc(c0 + 2), wexp_a, wsem_a)

            pltpu.make_async_copy(src_hbm.at[r_all.at[c1]], rows_b,
                                  gsem_b).wait()
            pltpu.make_async_copy(_wsrc(c1), wexp_b, wsem_b).wait()
            _scale_rows(rows_b, wexp_b)
            pltpu.sync_copy(rows_b, acc.at[c_all.at[c1]], add=True)

            @pl.when(c1 + 2 < _NCHUNK)
            def _():
                pltpu.async_copy(src_hbm.at[r_all.at[c1 + 2]], rows_b, gsem_b)
                pltpu.async_copy(_wsrc(c1 + 2), wexp_b, wsem_b)

            return carry

        lax.fori_loop(0, _NCHUNK // 2, chunk2, 0)
        plsc.subcore_barrier()
        pltpu.sync_copy(acc.at[pl.ds(sid * _RPT, _RPT)], out_hbm.at[cid, sid])
        plsc.subcore_barrier()


_agg_out = jax.ShapeDtypeStruct((_NC, _NS, _RPT, _GW), jnp.float32)
_sc_agg = pl.kernel(
    _sc_agg_body,
    out_type=[_agg_out, _agg_out, _agg_out],
    mesh=plsc.VectorSubcoreMesh(core_axis_name="c", subcore_axis_name="s"),
    scratch_types=[
        pltpu.VMEM((_NCHUNK, _K), jnp.int32),
        pltpu.VMEM((_NCHUNK, _K), jnp.int32),
        pltpu.VMEM((_WR, 128), jnp.float32),
        pltpu.VMEM((_WR, 128), jnp.float32),
        pltpu.VMEM((_K, _GW), jnp.float32),
        pltpu.VMEM((_K, _GW), jnp.float32),
        pltpu.VMEM((125, _GW), jnp.float32),
        pltpu.VMEM_SHARED((_N, _GW), jnp.float32),
        pltpu.SemaphoreType.DMA,
        pltpu.SemaphoreType.DMA,
        pltpu.SemaphoreType.DMA,
        pltpu.SemaphoreType.DMA,
    ],
    compiler_params=pltpu.CompilerParams(use_tc_tiling_on_sc=False),
)


_EPALL = _NT * _EPTP
_EB = _EPALL // 128 // 10


def _tc_wexp_body(ea_ref, m_ref, out_ref):
    w = ea_ref[...]
    w = jnp.abs(jnp.where(w == w, w, 0.0))
    out_ref[...] = jnp.dot(w, m_ref[...], preferred_element_type=jnp.float32)


def _tc_wexp(eap, m):
    return pl.pallas_call(
        _tc_wexp_body,
        grid=(10,),
        in_specs=[
            pl.BlockSpec((_EB, 128), lambda i: (i, 0)),
            pl.BlockSpec((128, 2048), lambda i: (0, 0)),
        ],
        out_specs=pl.BlockSpec((_EB, 2048), lambda i: (i, 0)),
        out_shape=jax.ShapeDtypeStruct((_EPALL // 128, 2048), jnp.float32),
    )(eap, m)


def _split_groups(ts):
    rows = ts.shape[0]
    return (ts[:, :_GW], ts[:, _GW:2 * _GW],
            jnp.concatenate(
                [ts[:, 2 * _GW:], jnp.zeros((rows, 3 * _GW - _H), jnp.float32)],
                axis=1))


def _merge_groups(ga, gb, gc):
    return jnp.concatenate([ga, gb, gc[:, :_H - 2 * _GW]], axis=1)


_SPEC_RBH = pl.BlockSpec((_RB, _H), lambda i: (i, 0))
_SPEC_RBG = pl.BlockSpec((_RB, _GW), lambda i: (i, 0))
_SPEC_RB1 = pl.BlockSpec((_RB, 1), lambda i: (i, 0))
_SPEC_1H = pl.BlockSpec((1, _H), lambda i: (0, 0))
_SPEC_HH = pl.BlockSpec((_H, _H), lambda i: (0, 0))

_SDS_G = jax.ShapeDtypeStruct((_N, _GW), jnp.float32)


def _tc_first_body(x_ref, w1a_ref, w1b_ref, d0_ref, d1_ref,
                   sa_ref, sb_ref, sc_ref, dis_ref):
    xb = x_ref[...]
    m = jnp.isnan(xb)
    xc = jnp.where(m, 0.0, xb)
    t = jnp.dot(xc, w1a_ref[...], preferred_element_type=jnp.float32)
    t = t + jnp.dot(m.astype(jnp.float32), w1b_ref[...],
                    preferred_element_type=jnp.float32)
    deg = d0_ref[...] + d1_ref[...] + 1.0
    dis = lax.rsqrt(deg)
    dis_ref[...] = dis
    ga, gb, gc = _split_groups(t * dis)
    sa_ref[...] = ga
    sb_ref[...] = gb
    sc_ref[...] = gc


def _tc_first(x, w1a, w1b, d0, d1):
    return pl.pallas_call(
        _tc_first_body,
        grid=(_NRB,),
        in_specs=[
            pl.BlockSpec((_RB, _D), lambda i: (i, 0)),
            pl.BlockSpec((_D, _H), lambda i: (0, 0)),
            pl.BlockSpec((_D, _H), lambda i: (0, 0)),
            _SPEC_RB1,
            _SPEC_RB1,
        ],
        out_specs=[_SPEC_RBG, _SPEC_RBG, _SPEC_RBG, _SPEC_RB1],
        out_shape=[_SDS_G, _SDS_G, _SDS_G,
                   jax.ShapeDtypeStruct((_N, 1), jnp.float32)],
    )(x, w1a, w1b, d0, d1)


def _tc_mid_body(a0a_ref, a1a_ref, a0b_ref, a1b_ref, a0c_ref, a1c_ref,
                 spa_ref, spb_ref, spc_ref, dis_ref, b_ref, g_ref, bb_ref,
                 w_ref, sa_ref, sb_ref, sc_ref):
    dis = dis_ref[...]
    agg = _merge_groups(a0a_ref[...] + a1a_ref[...] + spa_ref[...],
                        a0b_ref[...] + a1b_ref[...] + spb_ref[...],
                        a0c_ref[...] + a1c_ref[...] + spc_ref[...])
    u = agg * dis + b_ref[...]
    h = jnp.maximum(u * _KBN * g_ref[...] + bb_ref[...], 0.0)
    t = jnp.dot(h, w_ref[...], preferred_element_type=jnp.float32)
    ga, gb, gc = _split_groups(t * dis)
    sa_ref[...] = ga
    sb_ref[...] = gb
    sc_ref[...] = gc


def _tc_mid(agg, sp, dis, b, g, bb, w):
    return pl.pallas_call(
        _tc_mid_body,
        grid=(_NRB,),
        in_specs=[_SPEC_RBG] * 6 + [_SPEC_RBG] * 3 + [
            _SPEC_RB1, _SPEC_1H, _SPEC_1H, _SPEC_1H, _SPEC_HH,
        ],
        out_specs=[_SPEC_RBG, _SPEC_RBG, _SPEC_RBG],
        out_shape=[_SDS_G, _SDS_G, _SDS_G],
    )(agg[0][0], agg[0][1], agg[1][0], agg[1][1], agg[2][0], agg[2][1],
      sp[0], sp[1], sp[2], dis, b, g, bb, w)


def _tc_final_body(a0a_ref, a1a_ref, a0b_ref, a1b_ref, a0c_ref, a1c_ref,
                   spa_ref, spb_ref, spc_ref, dis_ref, b3_ref, batch_ref,
                   mw1_ref, mb1_ref, mw2_ref, mb2_ref, out_ref, sums, cnts):
    i = pl.program_id(0)

    @pl.when(i == 0)
    def _init():
        sums[...] = jnp.zeros_like(sums)
        cnts[...] = jnp.zeros_like(cnts)

    agg = _merge_groups(a0a_ref[...] + a1a_ref[...] + spa_ref[...],
                        a0b_ref[...] + a1b_ref[...] + spb_ref[...],
                        a0c_ref[...] + a1c_ref[...] + spc_ref[...])
    h3 = agg * dis_ref[...] + b3_ref[...]
    bv = batch_ref[0]
    oh = (lax.broadcasted_iota(jnp.int32, (_G, _RB), 0) == bv).astype(
        jnp.float32)
    sums[...] += jnp.dot(oh, h3, preferred_element_type=jnp.float32)
    cnts[...] = cnts[...] + jnp.sum(oh, axis=1, keepdims=True)

    @pl.when(i == pl.num_programs(0) - 1)
    def _fin():
        hg = sums[...] / jnp.maximum(cnts[...], 1.0)
        z1 = jnp.dot(hg, mw1_ref[...], preferred_element_type=jnp.float32)
        z1 = z1 + mb1_ref[...]
        z1 = 0.5 * z1 * (1.0 + lax.erf(z1 * float(1.0 / np.sqrt(2.0))))
        out_ref[...] = jnp.dot(z1, mw2_ref[...],
                               preferred_element_type=jnp.float32) + mb2_ref[...]


def _tc_final(agg, sp, dis, b3, batch3, mw1, mb1, mw2p, mb2p):
    return pl.pallas_call(
        _tc_final_body,
        grid=(_NRB,),
        in_specs=[_SPEC_RBG] * 6 + [_SPEC_RBG] * 3 + [
            _SPEC_RB1, _SPEC_1H,
            pl.BlockSpec((1, 1, _RB), lambda i: (i, 0, 0)),
            _SPEC_HH, _SPEC_1H, _SPEC_HH, _SPEC_1H,
        ],
        out_specs=pl.BlockSpec((_G, _H), lambda i: (0, 0)),
        out_shape=jax.ShapeDtypeStruct((_G, _H), jnp.float32),
        scratch_shapes=[
            pltpu.VMEM((_G, _H), jnp.float32),
            pltpu.VMEM((_G, _H), jnp.float32),
        ],
    )(agg[0][0], agg[0][1], agg[1][0], agg[1][1], agg[2][0], agg[2][1],
      sp[0], sp[1], sp[2], dis, b3, batch3, mw1, mb1, mw2p, mb2p)


def _agg_groups(sp, r3, c3, wn):
    outs = _sc_agg(sp[0], sp[1], sp[2], r3, c3, wn)
    res = []
    for o in outs:
        o = o.reshape(_NC, _N, _GW)
        res.append((o[0], o[1]))
    return res


def kernel(x, edge_index, edge_attr, batch, W1, b1, W2, b2, W3, b3,
           bn_g, bn_b, mW1, mb1, mW2, mb2):
    npad = _EPTP - _EPT
    cpad = jnp.broadcast_to((jnp.arange(npad, dtype=jnp.int32) * 41) % _N,
                            (_NT, npad))
    r3 = jnp.concatenate(
        [edge_index[0].astype(jnp.int32).reshape(_NT, _EPT), cpad],
        axis=1).reshape(_NT, _NCHUNK, _K)
    c3 = jnp.concatenate(
        [edge_index[1].astype(jnp.int32).reshape(_NT, _EPT), cpad],
        axis=1).reshape(_NT, _NCHUNK, _K)
    pad = ((0, 0), (0, npad))
    batch3 = batch.reshape(_NRB, 1, _RB).astype(jnp.int32)

    eap = jnp.pad(edge_attr.reshape(_NT, _EPT), pad).reshape(_EPALL // 128, 128)
    msel = (jnp.arange(128)[:, None] == (jnp.arange(2048) // 16)[None, :]
            ).astype(jnp.float32)
    wn = _tc_wexp(eap, msel).reshape(_NT, _EPTP // 8, 128)
    degp = _sc_deg(c3, wn).reshape(_NC, _N, 16)
    d0 = degp[0, :, 0:1]
    d1 = degp[1, :, 0:1]

    sa, sb, sc, dis = _tc_first(x, W1[:_D], W1[_D:], d0, d1)
    sp = (sa, sb, sc)

    b1r = b1.reshape(1, _H)
    b2r = b2.reshape(1, _H)
    b3r = b3.reshape(1, _H)
    gr = bn_g.reshape(1, _H)
    bbr = bn_b.reshape(1, _H)

    agg = _agg_groups(sp, r3, c3, wn)
    sp = _tc_mid(agg, sp, dis, b1r, gr, bbr, W2)
    agg = _agg_groups(sp, r3, c3, wn)
    sp = _tc_mid(agg, sp, dis, b2r, gr, bbr, W3)
    agg = _agg_groups(sp, r3, c3, wn)

    mw2p = jnp.pad(mW2, ((0, 0), (0, _H - _OUT)))
    mb2p = jnp.pad(mb2.reshape(1, _OUT), ((0, 0), (0, _H - _OUT)))
    zf = _tc_final(agg, sp, dis, b3r, batch3,
                   mW1, mb1.reshape(1, _H), mw2p, mb2p)
    return zf[:, :_OUT]


# speedup vs baseline: 2.6038x; 1.3688x over previous
import numpy as np
import jax
import jax.numpy as jnp
from jax import lax
from jax.experimental import pallas as pl
from jax.experimental.pallas import tpu as pltpu
from jax.experimental.pallas import tpu_sc as plsc

_N = 10000
_E = 320000
_D = 128
_H = 128
_OUT = 10
_G = 8
_NC = 2
_NS = 16
_NT = _NC * _NS
_EPT = _E // _NT
_K = 80
_EPTP = 10240
_NCHUNK = _EPTP // _K
_WR = _K // 8
_RPT = _N // _NS
_RB = 1000
_NRB = _N // _RB
_GW = 64
_NG = 2
_KBN = float(1.0 / np.sqrt(1.0 + 1e-5))


def _sc_deg_body(c_hbm, wn_hbm, out_hbm, c_all, wexp_a, wexp_b, wdeg_v, zbuf,
                 dacc, wsem_a, wsem_b):
    cid = lax.axis_index("c")
    sid = lax.axis_index("s")
    wid = cid * _NS + sid

    def zrow(i, carry):
        zbuf[i, :] = jnp.zeros((16,), jnp.float32)
        return carry

    lax.fori_loop(0, _RPT, zrow, 0)
    pltpu.sync_copy(zbuf, dacc.at[pl.ds(sid * _RPT, _RPT)])
    pltpu.sync_copy(c_hbm.at[wid], c_all)
    plsc.subcore_barrier()

    def _wsrc(i):
        return wn_hbm.at[wid, pl.ds(i * _WR, _WR)]

    def _expand(wexp_v):
        for k in range(_K):
            wdeg_v[k, :] = wexp_v[k // 8, pl.ds((k % 8) * 16, 16)]

    pltpu.async_copy(_wsrc(0), wexp_a, wsem_a)
    pltpu.async_copy(_wsrc(1), wexp_b, wsem_b)

    def chunk2(i2, carry):
        c0 = 2 * i2
        c1 = c0 + 1
        pltpu.make_async_copy(_wsrc(c0), wexp_a, wsem_a).wait()
        _expand(wexp_a)
        pltpu.sync_copy(wdeg_v, dacc.at[c_all.at[c0]], add=True)

        @pl.when(c0 + 2 < _NCHUNK)
        def _():
            pltpu.async_copy(_wsrc(c0 + 2), wexp_a, wsem_a)

        pltpu.make_async_copy(_wsrc(c1), wexp_b, wsem_b).wait()
        _expand(wexp_b)
        pltpu.sync_copy(wdeg_v, dacc.at[c_all.at[c1]], add=True)

        @pl.when(c1 + 2 < _NCHUNK)
        def _():
            pltpu.async_copy(_wsrc(c1 + 2), wexp_b, wsem_b)

        return carry

    lax.fori_loop(0, _NCHUNK // 2, chunk2, 0)
    plsc.subcore_barrier()
    pltpu.sync_copy(dacc.at[pl.ds(sid * _RPT, _RPT)], out_hbm.at[cid, sid])


_sc_deg = pl.kernel(
    _sc_deg_body,
    out_type=jax.ShapeDtypeStruct((_NC, _NS, _RPT, 16), jnp.float32),
    mesh=plsc.VectorSubcoreMesh(core_axis_name="c", subcore_axis_name="s"),
    scratch_types=[
        pltpu.VMEM((_NCHUNK, _K), jnp.int32),
        pltpu.VMEM((_WR, 128), jnp.float32),
        pltpu.VMEM((_WR, 128), jnp.float32),
        pltpu.VMEM((_K, 16), jnp.float32),
        pltpu.VMEM((_RPT, 16), jnp.float32),
        pltpu.VMEM_SHARED((_N, 16), jnp.float32),
        pltpu.SemaphoreType.DMA,
        pltpu.SemaphoreType.DMA,
    ],
    compiler_params=pltpu.CompilerParams(use_tc_tiling_on_sc=False),
)


def _scale_rows(rows_v, wexp_v):
    for k in range(_K):
        wb = wexp_v[k // 8, pl.ds((k % 8) * 16, 16)]
        for j in range(_GW // 16):
            sl = pl.ds(j * 16, 16)
            rows_v[k, sl] = rows_v[k, sl] * wb


def _sc_agg_body(sa_hbm, sb_hbm, r_hbm, c_hbm, wn_hbm,
                 oa_hbm, ob_hbm,
                 r_all, c_all, wexp_a, wexp_b, rows_a, rows_b, zbuf, acc,
                 gsem_a, gsem_b, wsem_a, wsem_b):
    cid = lax.axis_index("c")
    sid = lax.axis_index("s")
    wid = cid * _NS + sid

    pltpu.sync_copy(r_hbm.at[wid], r_all)
    pltpu.sync_copy(c_hbm.at[wid], c_all)

    def _wsrc(i):
        return wn_hbm.at[wid, pl.ds(i * _WR, _WR)]

    for src_hbm, out_hbm in ((sa_hbm, oa_hbm), (sb_hbm, ob_hbm)):
        def zrow(i, carry):
            for j in range(_GW // 16):
                zbuf[i, pl.ds(j * 16, 16)] = jnp.zeros((16,), jnp.float32)
            return carry

        lax.fori_loop(0, 125, zrow, 0)
        for q in range(_RPT // 125):
            pltpu.sync_copy(zbuf, acc.at[pl.ds(sid * _RPT + q * 125, 125)])
        plsc.subcore_barrier()

        pltpu.async_copy(src_hbm.at[r_all.at[0]], rows_a, gsem_a)
        pltpu.async_copy(_wsrc(0), wexp_a, wsem_a)
        pltpu.async_copy(src_hbm.at[r_all.at[1]], rows_b, gsem_b)
        pltpu.async_copy(_wsrc(1), wexp_b, wsem_b)

        def chunk2(i2, carry):
            c0 = 2 * i2
            c1 = c0 + 1
            pltpu.make_async_copy(src_hbm.at[r_all.at[c0]], rows_a,
                                  gsem_a).wait()
            pltpu.make_async_copy(_wsrc(c0), wexp_a, wsem_a).wait()
            _scale_rows(rows_a, wexp_a)
            pltpu.sync_copy(rows_a, acc.at[c_all.at[c0]], add=True)

            @pl.when(c0 + 2 < _NCHUNK)
            def _():
                pltpu.async_copy(src_hbm.at[r_all.at[c0 + 2]], rows_a, gsem_a)
                pltpu.async_copy(_wsrc(c0 + 2), wexp_a, wsem_a)

            pltpu.make_async_copy(src_hbm.at[r_all.at[c1]], rows_b,
                                  gsem_b).wait()
            pltpu.make_async_copy(_wsrc(c1), wexp_b, wsem_b).wait()
            _scale_rows(rows_b, wexp_b)
            pltpu.sync_copy(rows_b, acc.at[c_all.at[c1]], add=True)

            @pl.when(c1 + 2 < _NCHUNK)
            def _():
                pltpu.async_copy(src_hbm.at[r_all.at[c1 + 2]], rows_b, gsem_b)
                pltpu.async_copy(_wsrc(c1 + 2), wexp_b, wsem_b)

            return carry

        lax.fori_loop(0, _NCHUNK // 2, chunk2, 0)
        plsc.subcore_barrier()
        pltpu.sync_copy(acc.at[pl.ds(sid * _RPT, _RPT)], out_hbm.at[cid, sid])
        plsc.subcore_barrier()


_agg_out = jax.ShapeDtypeStruct((_NC, _NS, _RPT, _GW), jnp.float32)
_sc_agg = pl.kernel(
    _sc_agg_body,
    out_type=[_agg_out, _agg_out],
    mesh=plsc.VectorSubcoreMesh(core_axis_name="c", subcore_axis_name="s"),
    scratch_types=[
        pltpu.VMEM((_NCHUNK, _K), jnp.int32),
        pltpu.VMEM((_NCHUNK, _K), jnp.int32),
        pltpu.VMEM((_WR, 128), jnp.float32),
        pltpu.VMEM((_WR, 128), jnp.float32),
        pltpu.VMEM((_K, _GW), jnp.float32),
        pltpu.VMEM((_K, _GW), jnp.float32),
        pltpu.VMEM((125, _GW), jnp.float32),
        pltpu.VMEM_SHARED((_N, _GW), jnp.float32),
        pltpu.SemaphoreType.DMA,
        pltpu.SemaphoreType.DMA,
        pltpu.SemaphoreType.DMA,
        pltpu.SemaphoreType.DMA,
    ],
    compiler_params=pltpu.CompilerParams(use_tc_tiling_on_sc=False),
)


_EPALL = _NT * _EPTP
_EB = _EPALL // 128 // 10


def _tc_wexp_body(ea_ref, m_ref, out_ref):
    w = ea_ref[...]
    w = jnp.abs(jnp.where(w == w, w, 0.0))
    out_ref[...] = jnp.dot(w, m_ref[...], preferred_element_type=jnp.float32)


def _tc_wexp(eap, m):
    return pl.pallas_call(
        _tc_wexp_body,
        grid=(10,),
        in_specs=[
            pl.BlockSpec((_EB, 128), lambda i: (i, 0)),
            pl.BlockSpec((128, 2048), lambda i: (0, 0)),
        ],
        out_specs=pl.BlockSpec((_EB, 2048), lambda i: (i, 0)),
        out_shape=jax.ShapeDtypeStruct((_EPALL // 128, 2048), jnp.float32),
    )(eap, m)


def _split_groups(ts):
    return ts[:, :_GW], ts[:, _GW:]


def _merge_groups(ga, gb):
    return jnp.concatenate([ga, gb], axis=1)


_SPEC_RBH = pl.BlockSpec((_RB, _H), lambda i: (i, 0))
_SPEC_RBG = pl.BlockSpec((_RB, _GW), lambda i: (i, 0))
_SPEC_RB1 = pl.BlockSpec((_RB, 1), lambda i: (i, 0))
_SPEC_1H = pl.BlockSpec((1, _H), lambda i: (0, 0))
_SPEC_HH = pl.BlockSpec((_H, _H), lambda i: (0, 0))

_SDS_G = jax.ShapeDtypeStruct((_N, _GW), jnp.float32)


def _tc_first_body(x_ref, w1a_ref, w1b_ref, d0_ref, d1_ref,
                   sa_ref, sb_ref, dis_ref):
    xb = x_ref[...]
    m = jnp.isnan(xb)
    xc = jnp.where(m, 0.0, xb)
    t = jnp.dot(xc, w1a_ref[...], preferred_element_type=jnp.float32)
    t = t + jnp.dot(m.astype(jnp.float32), w1b_ref[...],
                    preferred_element_type=jnp.float32)
    deg = d0_ref[...] + d1_ref[...] + 1.0
    dis = lax.rsqrt(deg)
    dis_ref[...] = dis
    ga, gb = _split_groups(t * dis)
    sa_ref[...] = ga
    sb_ref[...] = gb


def _tc_first(x, w1a, w1b, d0, d1):
    return pl.pallas_call(
        _tc_first_body,
        grid=(_NRB,),
        in_specs=[
            pl.BlockSpec((_RB, _D), lambda i: (i, 0)),
            pl.BlockSpec((_D, _H), lambda i: (0, 0)),
            pl.BlockSpec((_D, _H), lambda i: (0, 0)),
            _SPEC_RB1,
            _SPEC_RB1,
        ],
        out_specs=[_SPEC_RBG, _SPEC_RBG, _SPEC_RB1],
        out_shape=[_SDS_G, _SDS_G,
                   jax.ShapeDtypeStruct((_N, 1), jnp.float32)],
    )(x, w1a, w1b, d0, d1)


def _tc_mid_body(a0a_ref, a1a_ref, a0b_ref, a1b_ref,
                 spa_ref, spb_ref, dis_ref, b_ref, g_ref, bb_ref,
                 w_ref, sa_ref, sb_ref):
    dis = dis_ref[...]
    agg = _merge_groups(a0a_ref[...] + a1a_ref[...] + spa_ref[...],
                        a0b_ref[...] + a1b_ref[...] + spb_ref[...])
    u = agg * dis + b_ref[...]
    h = jnp.maximum(u * _KBN * g_ref[...] + bb_ref[...], 0.0)
    t = jnp.dot(h, w_ref[...], preferred_element_type=jnp.float32)
    ga, gb = _split_groups(t * dis)
    sa_ref[...] = ga
    sb_ref[...] = gb


def _tc_mid(agg, sp, dis, b, g, bb, w):
    return pl.pallas_call(
        _tc_mid_body,
        grid=(_NRB,),
        in_specs=[_SPEC_RBG] * 4 + [_SPEC_RBG] * 2 + [
            _SPEC_RB1, _SPEC_1H, _SPEC_1H, _SPEC_1H, _SPEC_HH,
        ],
        out_specs=[_SPEC_RBG, _SPEC_RBG],
        out_shape=[_SDS_G, _SDS_G],
    )(agg[0][0], agg[0][1], agg[1][0], agg[1][1],
      sp[0], sp[1], dis, b, g, bb, w)


def _tc_final_body(a0a_ref, a1a_ref, a0b_ref, a1b_ref,
                   spa_ref, spb_ref, dis_ref, b3_ref, batch_ref,
                   mw1_ref, mb1_ref, mw2_ref, mb2_ref, out_ref, sums, cnts):
    i = pl.program_id(0)

    @pl.when(i == 0)
    def _init():
        sums[...] = jnp.zeros_like(sums)
        cnts[...] = jnp.zeros_like(cnts)

    agg = _merge_groups(a0a_ref[...] + a1a_ref[...] + spa_ref[...],
                        a0b_ref[...] + a1b_ref[...] + spb_ref[...])
    h3 = agg * dis_ref[...] + b3_ref[...]
    bv = batch_ref[0]
    oh = (lax.broadcasted_iota(jnp.int32, (_G, _RB), 0) == bv).astype(
        jnp.float32)
    sums[...] += jnp.dot(oh, h3, preferred_element_type=jnp.float32)
    cnts[...] = cnts[...] + jnp.sum(oh, axis=1, keepdims=True)

    @pl.when(i == pl.num_programs(0) - 1)
    def _fin():
        hg = sums[...] / jnp.maximum(cnts[...], 1.0)
        z1 = jnp.dot(hg, mw1_ref[...], preferred_element_type=jnp.float32)
        z1 = z1 + mb1_ref[...]
        z1 = 0.5 * z1 * (1.0 + lax.erf(z1 * float(1.0 / np.sqrt(2.0))))
        out_ref[...] = jnp.dot(z1, mw2_ref[...],
                               preferred_element_type=jnp.float32) + mb2_ref[...]


def _tc_final(agg, sp, dis, b3, batch3, mw1, mb1, mw2p, mb2p):
    return pl.pallas_call(
        _tc_final_body,
        grid=(_NRB,),
        in_specs=[_SPEC_RBG] * 4 + [_SPEC_RBG] * 2 + [
            _SPEC_RB1, _SPEC_1H,
            pl.BlockSpec((1, 1, _RB), lambda i: (i, 0, 0)),
            _SPEC_HH, _SPEC_1H, _SPEC_HH, _SPEC_1H,
        ],
        out_specs=pl.BlockSpec((_G, _H), lambda i: (0, 0)),
        out_shape=jax.ShapeDtypeStruct((_G, _H), jnp.float32),
        scratch_shapes=[
            pltpu.VMEM((_G, _H), jnp.float32),
            pltpu.VMEM((_G, _H), jnp.float32),
        ],
    )(agg[0][0], agg[0][1], agg[1][0], agg[1][1],
      sp[0], sp[1], dis, b3, batch3, mw1, mb1, mw2p, mb2p)


def _agg_groups(sp, r3, c3, wn):
    outs = _sc_agg(sp[0], sp[1], r3, c3, wn)
    res = []
    for o in outs:
        o = o.reshape(_NC, _N, _GW)
        res.append((o[0], o[1]))
    return res


def kernel(x, edge_index, edge_attr, batch, W1, b1, W2, b2, W3, b3,
           bn_g, bn_b, mW1, mb1, mW2, mb2):
    npad = _EPTP - _EPT
    cpad = jnp.broadcast_to((jnp.arange(npad, dtype=jnp.int32) * 41) % _N,
                            (_NT, npad))
    r3 = jnp.concatenate(
        [edge_index[0].astype(jnp.int32).reshape(_NT, _EPT), cpad],
        axis=1).reshape(_NT, _NCHUNK, _K)
    c3 = jnp.concatenate(
        [edge_index[1].astype(jnp.int32).reshape(_NT, _EPT), cpad],
        axis=1).reshape(_NT, _NCHUNK, _K)
    pad = ((0, 0), (0, npad))
    batch3 = batch.reshape(_NRB, 1, _RB).astype(jnp.int32)

    eap = jnp.pad(edge_attr.reshape(_NT, _EPT), pad).reshape(_EPALL // 128, 128)
    msel = (jnp.arange(128)[:, None] == (jnp.arange(2048) // 16)[None, :]
            ).astype(jnp.float32)
    wn = _tc_wexp(eap, msel).reshape(_NT, _EPTP // 8, 128)
    degp = _sc_deg(c3, wn).reshape(_NC, _N, 16)
    d0 = degp[0, :, 0:1]
    d1 = degp[1, :, 0:1]

    sa, sb, dis = _tc_first(x, W1[:_D], W1[_D:], d0, d1)
    sp = (sa, sb)

    b1r = b1.reshape(1, _H)
    b2r = b2.reshape(1, _H)
    b3r = b3.reshape(1, _H)
    gr = bn_g.reshape(1, _H)
    bbr = bn_b.reshape(1, _H)

    agg = _agg_groups(sp, r3, c3, wn)
    sp = _tc_mid(agg, sp, dis, b1r, gr, bbr, W2)
    agg = _agg_groups(sp, r3, c3, wn)
    sp = _tc_mid(agg, sp, dis, b2r, gr, bbr, W3)
    agg = _agg_groups(sp, r3, c3, wn)

    mw2p = jnp.pad(mW2, ((0, 0), (0, _H - _OUT)))
    mb2p = jnp.pad(mb2.reshape(1, _OUT), ((0, 0), (0, _H - _OUT)))
    zf = _tc_final(agg, sp, dis, b3r, batch3,
                   mW1, mb1.reshape(1, _H), mw2p, mb2p)
    return zf[:, :_OUT]
